# Initial kernel scaffold; baseline (speedup 1.0000x reference)
import jax
import jax.numpy as jnp
from jax.experimental import pallas as pl


def kernel(x, edge_index, W, B):
    raise NotImplementedError("write your pallas kernel here")


# speedup vs baseline: 4.4793x; 4.4793x over previous
import functools

import jax
import jax.numpy as jnp
from jax import lax
from jax.experimental import pallas as pl
from jax.experimental.pallas import tpu as pltpu
from jax.experimental.pallas import tpu_sc as plsc

N = 10000
E = 160000
F = 256
H = 128
PW = 144
NC = 2
NS = 16
ET = E // NS
GB = 80
NG = ET // GB
NP = 10240
RT = NP // NS
ZC = 64
IC = 5
IG = NG // IC
BM = 1000


def _mm_body(x_ref, w_ref, p_ref):
    y = lax.dot_general(x_ref[...], w_ref[...],
                        (((1,), (1,)), ((), ())),
                        preferred_element_type=jnp.float32)
    pad = jnp.concatenate(
        [jnp.ones((BM, 1), jnp.float32), jnp.zeros((BM, PW - H - 1), jnp.float32)],
        axis=1)
    p_ref[0] = jnp.concatenate([y[:, :H], pad], axis=1)
    p_ref[1] = jnp.concatenate([y[:, H:], pad], axis=1)


def _mm(x, W):
    return pl.pallas_call(
        _mm_body,
        grid=(N // BM,),
        in_specs=[
            pl.BlockSpec((BM, F), lambda i: (i, 0)),
            pl.BlockSpec((F, F), lambda i: (0, 0)),
        ],
        out_specs=pl.BlockSpec((NC, BM, PW), lambda i: (0, i, 0)),
        out_shape=jax.ShapeDtypeStruct((NC, N, PW), jnp.float32),
    )(x, W)


def _sc_agg_body(p_hbm, src_hbm, dst_hbm, acc_hbm,
                 src_v, dst_v, rows_v, zbuf_v, acc_sh, sem):
    c = lax.axis_index("c")
    s = lax.axis_index("s")

    def zero_row(r, _):
        for k in range(PW // 16):
            zbuf_v[r, pl.ds(k * 16, 16)] = jnp.zeros((16,), jnp.float32)
        return 0
    lax.fori_loop(0, ZC, zero_row, 0, unroll=False)
    for q in range(RT // ZC):
        pltpu.sync_copy(zbuf_v, acc_sh.at[pl.ds(s * RT + q * ZC, ZC)])
    plsc.subcore_barrier()

    off = c * N
    for q in range(IC):
        pltpu.sync_copy(src_hbm.at[s, pl.ds(q * IG, IG)], src_v)
        pltpu.sync_copy(dst_hbm.at[s, pl.ds(q * IG, IG)], dst_v)

        def add_off(r, _):
            for k in range(GB // 16):
                v = src_v[r, pl.ds(k * 16, 16)]
                src_v[r, pl.ds(k * 16, 16)] = v + off
            return 0
        lax.fori_loop(0, IG, add_off, 0, unroll=False)

        def edge_group(g, _):
            pltpu.async_copy(p_hbm.at[src_v.at[g]], rows_v, sem).wait()
            pltpu.sync_copy(rows_v, acc_sh.at[dst_v.at[g]], add=True)
            return 0
        lax.fori_loop(0, IG, edge_group, 0, unroll=False)

    plsc.subcore_barrier()
    pltpu.sync_copy(acc_sh.at[pl.ds(s * RT, RT)],
                    acc_hbm.at[c, pl.ds(s * RT, RT)])


@functools.partial(
    pl.kernel,
    out_type=jax.ShapeDtypeStruct((NC, NP, PW), jnp.float32),
    mesh=plsc.VectorSubcoreMesh(core_axis_name="c", subcore_axis_name="s"),
    scratch_types=[
        pltpu.VMEM((IG, GB), jnp.int32),
        pltpu.VMEM((IG, GB), jnp.int32),
        pltpu.VMEM((GB, PW), jnp.float32),
        pltpu.VMEM((ZC, PW), jnp.float32),
        pltpu.VMEM_SHARED((NP, PW), jnp.float32),
        pltpu.SemaphoreType.DMA,
    ],
    compiler_params=pltpu.CompilerParams(use_tc_tiling_on_sc=False),
)
def _sc_agg(*refs):
    _sc_agg_body(*refs)


def _finish_body(acc_ref, x_ref, b_ref, o_ref):
    a0 = acc_ref[0]
    a1 = acc_ref[1]
    c0 = jnp.maximum(a0[:, H:H + 1], 1.0)
    c1 = jnp.maximum(a1[:, H:H + 1], 1.0)
    neigh = jnp.concatenate([a0[:, :H] / c0, a1[:, :H] / c1], axis=1)
    self_t = lax.dot_general(x_ref[...], b_ref[...],
                             (((1,), (1,)), ((), ())),
                             preferred_element_type=jnp.float32)
    o_ref[...] = neigh + self_t


def _finish(acc, x, B):
    return pl.pallas_call(
        _finish_body,
        grid=(N // BM,),
        in_specs=[
            pl.BlockSpec((NC, BM, PW), lambda i: (0, i, 0)),
            pl.BlockSpec((BM, F), lambda i: (i, 0)),
            pl.BlockSpec((F, F), lambda i: (0, 0)),
        ],
        out_specs=pl.BlockSpec((BM, F), lambda i: (i, 0)),
        out_shape=jax.ShapeDtypeStruct((N, F), jnp.float32),
    )(acc, x, B)


def kernel(x, edge_index, W, B):
    src = edge_index[0].astype(jnp.int32).reshape(NS, NG, GB)
    dst = edge_index[1].astype(jnp.int32).reshape(NS, NG, GB)
    p = _mm(x, W).reshape(NC * N, PW)
    acc = _sc_agg(p, src, dst)
    return _finish(acc, x, B)
